# baseline (speedup 1.0000x reference)
import functools

import jax
import jax.numpy as jnp
from jax import lax
from jax.experimental import pallas as pl
from jax.experimental.pallas import tpu as pltpu
from jax.experimental.pallas import tpu_sc as plsc

_WIN = 128
_WG = 1
_NBUF = 14
_GRP = 128


@functools.lru_cache(maxsize=None)
def _build(B, V, D):
    info = plsc.get_sparse_core_info()
    nw = info.num_cores * info.num_subcores
    b_per_w = B // nw
    assert b_per_w * nw == B and b_per_w % _GRP == 0
    n_grp = b_per_w // _GRP
    da = D // 8
    mesh = plsc.VectorSubcoreMesh(core_axis_name="c", subcore_axis_name="s")

    @functools.partial(
        pl.kernel,
        mesh=mesh,
        out_type=jax.ShapeDtypeStruct((da, 8, B), jnp.float32),
        scratch_types=[
            pltpu.VMEM((b_per_w,), jnp.int32),
            pltpu.VMEM((b_per_w,), jnp.int32),
            pltpu.VMEM((_NBUF, _WG, da, 8, _WIN), jnp.float32),
            pltpu.VMEM((da, 8, _GRP), jnp.float32),
        ] + [pltpu.SemaphoreType.DMA] * (_NBUF + 1),
        compiler_params=pltpu.CompilerParams(
            needs_layout_passes=False, disable_bounds_checks=True),
    )
    def gather_kernel(table_hbm, idx_hbm, out_hbm, idx_v, col_v, gbuf, obuf, *sems):
        wid = lax.axis_index("s") * info.num_cores + lax.axis_index("c")
        base = wid * b_per_w
        pltpu.sync_copy(idx_hbm.at[wid], idx_v)

        for k in range(b_per_w // 16):
            v = idx_v[pl.ds(k * 16, 16)]
            off = (v >> 7) << 7
            idx_v[pl.ds(k * 16, 16)] = off
            col_v[pl.ds(k * 16, 16)] = v - off

        jota = lax.iota(jnp.int32, 16)
        avec = [(jota + k * 16) >> 3 for k in range(da // 2)]
        bvec = (jota & 7).astype(jnp.int32)

        ones = jnp.full((16,), 1, jnp.int32)
        osem = sems[-1]
        n_fg = _GRP // _WG

        def grp_body(g, carry):
            offs = [idx_v[pl.ds(g * _GRP + q * 16, 16)]
                    for q in range(_GRP // 16)]
            cols = [col_v[pl.ds(g * _GRP + q * 16, 16)]
                    for q in range(_GRP // 16)]

            def fire(s):
                buf = s % _NBUF
                out = []
                for l in range(_WG):
                    lane = s * _WG + l
                    o = pl.multiple_of(offs[lane // 16][lane % 16], _WIN)
                    out.append(pltpu.async_copy(
                        table_hbm.at[:, :, pl.ds(o, _WIN)],
                        gbuf.at[buf, l], sems[buf],
                    ))
                return out

            def select(s, copies):
                buf = s % _NBUF
                bsp = jnp.full((16,), buf, jnp.int32)
                for c in copies:
                    c.wait()
                for l in range(_WG):
                    lane = s * _WG + l
                    csp = ones * cols[lane // 16][lane % 16]
                    ssp = jnp.full((16,), l, jnp.int32)
                    isp = jnp.full((16,), lane, jnp.int32)
                    for k in range(da // 2):
                        vals = plsc.load_gather(
                            gbuf, [bsp, ssp, avec[k], bvec, csp])
                        plsc.store_scatter(
                            obuf, [avec[k], bvec, isp], vals)

            ahead = _NBUF - 1
            pend = [fire(t) for t in range(ahead)]
            for s in range(n_fg):
                if s + ahead < n_fg:
                    pend.append(fire(s + ahead))
                select(s, pend.pop(0))
            pltpu.async_copy(
                obuf, out_hbm.at[:, :, pl.ds(base + g * _GRP, _GRP)], osem,
            ).wait()
            return carry

        lax.fori_loop(0, n_grp, grp_body, 0)

    return gather_kernel


def kernel(input, action_embedding):
    B = input.shape[0]
    V, D = action_embedding.shape
    idx = input.reshape(B).astype(jnp.int32)
    info = plsc.get_sparse_core_info()
    nw = info.num_cores * info.num_subcores
    idx2 = idx.reshape(nw, B // nw)
    table_t = action_embedding.T.reshape(D // 8, 8, V)
    out_t = _build(B, V, D)(table_t, idx2)
    return out_t.reshape(D, B).T


# speedup vs baseline: 1.0435x; 1.0435x over previous
import functools

import jax
import jax.numpy as jnp
from jax import lax
from jax.experimental import pallas as pl
from jax.experimental.pallas import tpu as pltpu
from jax.experimental.pallas import tpu_sc as plsc

_WIN = 128
_WG = 2
_GRP = 128


@functools.lru_cache(maxsize=None)
def _build(B, V, D):
    info = plsc.get_sparse_core_info()
    nw = info.num_cores * info.num_subcores
    b_per_w = B // nw
    assert b_per_w * nw == B and b_per_w % _GRP == 0
    n_grp = b_per_w // _GRP
    da = D // 8
    mesh = plsc.VectorSubcoreMesh(core_axis_name="c", subcore_axis_name="s")

    @functools.partial(
        pl.kernel,
        mesh=mesh,
        out_type=jax.ShapeDtypeStruct((da, 8, B), jnp.float32),
        scratch_types=[
            pltpu.VMEM((b_per_w,), jnp.int32),
            pltpu.VMEM((b_per_w,), jnp.int32),
            pltpu.VMEM((7, _WG, da, 8, _WIN), jnp.float32),
            pltpu.VMEM((da, 8, _GRP), jnp.float32),
        ] + [pltpu.SemaphoreType.DMA] * 8,
        compiler_params=pltpu.CompilerParams(
            needs_layout_passes=False, disable_bounds_checks=True),
    )
    def gather_kernel(table_hbm, idx_hbm, out_hbm, idx_v, col_v, gbuf, obuf, *sems):
        wid = lax.axis_index("s") * info.num_cores + lax.axis_index("c")
        base = wid * b_per_w
        pltpu.sync_copy(idx_hbm.at[wid], idx_v)

        for k in range(b_per_w // 16):
            v = idx_v[pl.ds(k * 16, 16)]
            off = (v >> 7) << 7
            idx_v[pl.ds(k * 16, 16)] = off
            col_v[pl.ds(k * 16, 16)] = v - off

        jota = lax.iota(jnp.int32, 16)
        avec = [(jota + k * 16) >> 3 for k in range(da // 2)]
        bvec = (jota & 7).astype(jnp.int32)

        ones = jnp.full((16,), 1, jnp.int32)
        osem = sems[-1]
        n_fg = _GRP // _WG

        def grp_body(g, carry):
            offs = [idx_v[pl.ds(g * _GRP + q * 16, 16)]
                    for q in range(_GRP // 16)]
            cols = [col_v[pl.ds(g * _GRP + q * 16, 16)]
                    for q in range(_GRP // 16)]

            def fire(s):
                buf = s % 7
                out = []
                for l in range(_WG):
                    lane = s * _WG + l
                    o = pl.multiple_of(offs[lane // 16][lane % 16], _WIN)
                    out.append(pltpu.async_copy(
                        table_hbm.at[:, :, pl.ds(o, _WIN)],
                        gbuf.at[buf, l], sems[buf],
                    ))
                return out

            def select(s, copies):
                buf = s % 7
                bsp = jnp.full((16,), buf, jnp.int32)
                for c in copies:
                    c.wait()
                for l in range(_WG):
                    lane = s * _WG + l
                    csp = ones * cols[lane // 16][lane % 16]
                    ssp = jnp.full((16,), l, jnp.int32)
                    isp = jnp.full((16,), lane, jnp.int32)
                    for k in range(da // 2):
                        vals = plsc.load_gather(
                            gbuf, [bsp, ssp, avec[k], bvec, csp])
                        plsc.store_scatter(
                            obuf, [avec[k], bvec, isp], vals)

            pend = [fire(t) for t in range(6)]
            for s in range(n_fg):
                if s + 6 < n_fg:
                    pend.append(fire(s + 6))
                select(s, pend.pop(0))
            pltpu.async_copy(
                obuf, out_hbm.at[:, :, pl.ds(base + g * _GRP, _GRP)], osem,
            ).wait()
            return carry

        lax.fori_loop(0, n_grp, grp_body, 0)

    return gather_kernel


def kernel(input, action_embedding):
    B = input.shape[0]
    V, D = action_embedding.shape
    idx = input.reshape(B).astype(jnp.int32)
    info = plsc.get_sparse_core_info()
    nw = info.num_cores * info.num_subcores
    idx2 = idx.reshape(nw, B // nw)
    table_t = action_embedding.T.reshape(D // 8, 8, V)
    out_t = _build(B, V, D)(table_t, idx2)
    return out_t.reshape(D, B).T
